# baseline (speedup 1.0000x reference)
import jax
import jax.numpy as jnp
from jax import lax
from jax.experimental import pallas as pl
from jax.experimental.pallas import tpu as pltpu
from jax.experimental.pallas import tpu_sc as plsc

NC = 2
NS = 16
L = 16
NW = NC * NS

B = 16384
D = 64
V = 1000000

SLABW = 256
NSLAB = V // SLABW
Z2R = 500096

BPW = B // NW
CH = 128
NCH = BPW // CH


def _mesh():
    return plsc.VectorSubcoreMesh(core_axis_name="c", subcore_axis_name="s",
                                  num_cores=NC, num_subcores=NS)


def _tr_body(zt_hbm, ztail_hbm, z2_hbm, slab0, slab1, out0, out1, tailv,
             sem_r0, sem_r1, sem_w0, sem_w1):
    wid = lax.axis_index("s") * NC + lax.axis_index("c")
    lanes = lax.iota(jnp.int32, L)

    rowidx = [(c * L + lanes) >> 1 for c in range(SLABW // L)]
    colbase = [((c * L + lanes) & 1) * D for c in range(SLABW // L)]

    def fire_read(s, slab, sem):
        off = pl.multiple_of(s * SLABW, 128)
        return pltpu.async_copy(zt_hbm.at[:, pl.ds(off, SLABW)], slab, sem)

    def drain(sem):
        pltpu.make_async_copy(z2_hbm.at[pl.ds(0, SLABW // 2), :], out0,
                              sem).wait()

    def transpose(slab, out):
        @plsc.parallel_loop(0, D, step=1, unroll=8)
        def _(f):
            for c in range(SLABW // L):
                v = slab[f, pl.ds(c * L, L)]
                plsc.store_scatter(out, [rowidx[c], colbase[c] + f], v)

    def fire_write(s, out, sem):
        pltpu.async_copy(out, z2_hbm.at[pl.ds(s * (SLABW // 2), SLABW // 2), :],
                         sem)

    fire_read(wid, slab0, sem_r0)

    def step(i, carry):
        m0 = 2 * i
        s0 = wid + 32 * m0
        s1 = s0 + 32

        fire_read(s1, slab1, sem_r1)
        drain(sem_r0)

        @pl.when(i > 0)
        def _():
            drain(sem_w0)

        transpose(slab0, out0)
        fire_write(s0, out0, sem_w0)

        @pl.when(i <= 59)
        def _():
            fire_read(s1 + 32, slab0, sem_r0)

        drain(sem_r1)

        @pl.when(i > 0)
        def _():
            drain(sem_w1)

        transpose(slab1, out1)
        fire_write(s1, out1, sem_w1)
        return carry

    lax.fori_loop(0, 61, step, 0)

    s_tail = wid + 32 * 122

    @pl.when(wid <= 1)
    def _():
        fire_read(s_tail, slab0, sem_r0)
        drain(sem_r0)
        drain(sem_w0)
        transpose(slab0, out0)
        fire_write(s_tail, out0, sem_w0)

    @pl.when(wid == 5)
    def _():
        pltpu.sync_copy(ztail_hbm, tailv)
        drain(sem_w0)
        for r in range(64):
            for c in range(4):
                v = tailv[pl.ds(r * D + c * L, L)]
                out0[r // 2, pl.ds((r % 2) * D + c * L, L)] = v
        fire_write(NSLAB, out0, sem_w0)

    drain(sem_w1)
    drain(sem_w0)


def _transpose(zt, ztail):
    return pl.kernel(
        _tr_body,
        out_type=jax.ShapeDtypeStruct((Z2R, 128), jnp.float32),
        mesh=_mesh(),
        compiler_params=pltpu.CompilerParams(needs_layout_passes=False),
        scratch_types=[
            pltpu.VMEM((D, SLABW), jnp.float32),
            pltpu.VMEM((D, SLABW), jnp.float32),
            pltpu.VMEM((SLABW // 2, 128), jnp.float32),
            pltpu.VMEM((SLABW // 2, 128), jnp.float32),
            pltpu.VMEM((64 * D,), jnp.float32),
            pltpu.SemaphoreType.DMA,
            pltpu.SemaphoreType.DMA,
            pltpu.SemaphoreType.DMA,
            pltpu.SemaphoreType.DMA,
        ],
    )(zt, ztail)


def _decoder_body(z2_hbm, h_hbm, t_hbm, out_hbm,
                  idx_v, gat_v, par_v, hrows, trows, out_v, sem):
    wid = lax.axis_index("s") * NC + lax.axis_index("c")
    base = wid * BPW

    pltpu.sync_copy(h_hbm.at[pl.ds(base, BPW)], idx_v.at[pl.ds(0, BPW)])
    pltpu.sync_copy(t_hbm.at[pl.ds(base, BPW)], idx_v.at[pl.ds(BPW, BPW)])

    for k in range(2 * BPW // L):
        v = idx_v[pl.ds(k * L, L)]
        gat_v[k * L // CH, pl.ds((k * L) % CH, L)] = v >> 1
        par_v[pl.ds(k * L, L)] = v & 1

    def fire(k, buf):
        return (
            pltpu.async_copy(z2_hbm.at[gat_v.at[k]], hrows.at[buf], sem),
            pltpu.async_copy(z2_hbm.at[gat_v.at[NCH + k]], trows.at[buf], sem),
        )

    lanes = lax.iota(jnp.int32, L)

    def compute(k, buf):
        def group(g, carry):
            ph = par_v[pl.ds(k * CH + g * L, L)] > 0
            pt = par_v[pl.ds(BPW + k * CH + g * L, L)] > 0
            rll = jnp.zeros((L,), jnp.float32)
            rlr = jnp.zeros((L,), jnp.float32)
            rrl = jnp.zeros((L,), jnp.float32)
            rrr = jnp.zeros((L,), jnp.float32)
            for j in range(L):
                r = g * L + j
                sll = jnp.zeros((L,), jnp.float32)
                slr = jnp.zeros((L,), jnp.float32)
                srl = jnp.zeros((L,), jnp.float32)
                srr = jnp.zeros((L,), jnp.float32)
                for c in range(D // L):
                    hlo = hrows[buf, r, pl.ds(c * L, L)]
                    hhi = hrows[buf, r, pl.ds(D + c * L, L)]
                    tlo = trows[buf, r, pl.ds(c * L, L)]
                    thi = trows[buf, r, pl.ds(D + c * L, L)]
                    sll = sll + hlo * tlo
                    slr = slr + hlo * thi
                    srl = srl + hhi * tlo
                    srr = srr + hhi * thi
                m = lanes == j
                rll = jnp.where(m, jnp.sum(sll), rll)
                rlr = jnp.where(m, jnp.sum(slr), rlr)
                rrl = jnp.where(m, jnp.sum(srl), rrl)
                rrr = jnp.where(m, jnp.sum(srr), rrr)
            res = jnp.where(ph, jnp.where(pt, rrr, rrl),
                            jnp.where(pt, rlr, rll))
            out_v[pl.ds(k * CH + g * L, L)] = res
            return carry

        lax.fori_loop(0, CH // L, group, 0)

    descs = {0: fire(0, 0)}
    for k in range(NCH):
        if k + 1 < NCH:
            descs[k + 1] = fire(k + 1, (k + 1) % 2)
        for d in descs.pop(k):
            d.wait()
        compute(k, k % 2)

    pltpu.sync_copy(out_v, out_hbm.at[pl.ds(base, BPW)])


def _decode(z2, h, t):
    return pl.kernel(
        _decoder_body,
        out_type=jax.ShapeDtypeStruct((B,), jnp.float32),
        mesh=_mesh(),
        compiler_params=pltpu.CompilerParams(needs_layout_passes=False),
        scratch_types=[
            pltpu.VMEM((2 * BPW,), jnp.int32),
            pltpu.VMEM((2 * NCH, CH), jnp.int32),
            pltpu.VMEM((2 * BPW,), jnp.int32),
            pltpu.VMEM((2, CH, 2 * D), jnp.float32),
            pltpu.VMEM((2, CH, 2 * D), jnp.float32),
            pltpu.VMEM((BPW,), jnp.float32),
            pltpu.SemaphoreType.DMA,
        ],
    )(z2, h, t)


def kernel(z, triples):
    h = triples[:, 0].astype(jnp.int32)
    t = triples[:, 2].astype(jnp.int32)
    ztail = z[NSLAB * SLABW:].reshape(-1)
    z2 = _transpose(z.T, ztail)
    return _decode(z2, h, t)


# speedup vs baseline: 3.9186x; 3.9186x over previous
import jax
import jax.numpy as jnp
from jax import lax
from jax.experimental import pallas as pl
from jax.experimental.pallas import tpu as pltpu
from jax.experimental.pallas import tpu_sc as plsc

NC = 2
NS = 16
L = 16
NW = NC * NS

B = 16384
D = 64
V = 1000000

SLABW = 256
NSLAB = V // SLABW
Z2R = 500096

BPW = B // NW
CH = 128
NCH = BPW // CH


def _mesh():
    return plsc.VectorSubcoreMesh(core_axis_name="c", subcore_axis_name="s",
                                  num_cores=NC, num_subcores=NS)


def _tr_body(zt_hbm, ztail_hbm, z2_hbm, slab0, slab1, out0, out1, tailv,
             pad_v, sem_r0, sem_r1, sem_w0, sem_w1):
    wid = lax.axis_index("s") * NC + lax.axis_index("c")
    lanes = lax.iota(jnp.int32, L)

    rowbase = [((c * L + lanes) >> 1) * 130 + ((c * L + lanes) & 1) * 65
               for c in range(SLABW // L)]

    def fire_read(s, slab, sem):
        off = pl.multiple_of(s * SLABW, 128)
        return pltpu.async_copy(zt_hbm.at[:, pl.ds(off, SLABW)], slab, sem)

    def drain(sem):
        pltpu.make_async_copy(z2_hbm.at[pl.ds(0, SLABW // 2), :], out0,
                              sem).wait()

    def transpose(slab, out):
        @plsc.parallel_loop(0, D, step=1, unroll=8)
        def _(f):
            for c in range(SLABW // L):
                v = slab[f, pl.ds(c * L, L)]
                plsc.store_scatter(pad_v, [rowbase[c] + f], v)

        @plsc.parallel_loop(0, SLABW // 2, step=1, unroll=8)
        def _(k):
            for c2 in range(8):
                off = k * 130 + (65 if c2 >= 4 else 0) + (c2 % 4) * L
                out[k, pl.ds(c2 * L, L)] = pad_v[pl.ds(off, L)]

    def fire_write(s, out, sem):
        pltpu.async_copy(out, z2_hbm.at[pl.ds(s * (SLABW // 2), SLABW // 2), :],
                         sem)

    fire_read(wid, slab0, sem_r0)

    def step(i, carry):
        m0 = 2 * i
        s0 = wid + 32 * m0
        s1 = s0 + 32

        fire_read(s1, slab1, sem_r1)
        drain(sem_r0)

        @pl.when(i > 0)
        def _():
            drain(sem_w0)

        transpose(slab0, out0)
        fire_write(s0, out0, sem_w0)

        @pl.when(i <= 59)
        def _():
            fire_read(s1 + 32, slab0, sem_r0)

        drain(sem_r1)

        @pl.when(i > 0)
        def _():
            drain(sem_w1)

        transpose(slab1, out1)
        fire_write(s1, out1, sem_w1)
        return carry

    lax.fori_loop(0, 61, step, 0)

    s_tail = wid + 32 * 122

    @pl.when(wid <= 1)
    def _():
        fire_read(s_tail, slab0, sem_r0)
        drain(sem_r0)
        drain(sem_w0)
        transpose(slab0, out0)
        fire_write(s_tail, out0, sem_w0)

    @pl.when(wid == 5)
    def _():
        pltpu.sync_copy(ztail_hbm, tailv)
        drain(sem_w0)
        for r in range(64):
            for c in range(4):
                v = tailv[pl.ds(r * D + c * L, L)]
                out0[r // 2, pl.ds((r % 2) * D + c * L, L)] = v
        fire_write(NSLAB, out0, sem_w0)

    drain(sem_w1)
    drain(sem_w0)


def _transpose(zt, ztail):
    return pl.kernel(
        _tr_body,
        out_type=jax.ShapeDtypeStruct((Z2R, 128), jnp.float32),
        mesh=_mesh(),
        compiler_params=pltpu.CompilerParams(needs_layout_passes=False),
        scratch_types=[
            pltpu.VMEM((D, SLABW), jnp.float32),
            pltpu.VMEM((D, SLABW), jnp.float32),
            pltpu.VMEM((SLABW // 2, 128), jnp.float32),
            pltpu.VMEM((SLABW // 2, 128), jnp.float32),
            pltpu.VMEM((64 * D,), jnp.float32),
            pltpu.VMEM(((SLABW // 2) * 130,), jnp.float32),
            pltpu.SemaphoreType.DMA,
            pltpu.SemaphoreType.DMA,
            pltpu.SemaphoreType.DMA,
            pltpu.SemaphoreType.DMA,
        ],
    )(zt, ztail)


def _decoder_body(z2_hbm, h_hbm, t_hbm, out_hbm,
                  idx_v, gat_v, par_v, hrows, trows, out_v, sem):
    wid = lax.axis_index("s") * NC + lax.axis_index("c")
    base = wid * BPW

    pltpu.sync_copy(h_hbm.at[pl.ds(base, BPW)], idx_v.at[pl.ds(0, BPW)])
    pltpu.sync_copy(t_hbm.at[pl.ds(base, BPW)], idx_v.at[pl.ds(BPW, BPW)])

    for k in range(2 * BPW // L):
        v = idx_v[pl.ds(k * L, L)]
        gat_v[k * L // CH, pl.ds((k * L) % CH, L)] = v >> 1
        par_v[pl.ds(k * L, L)] = v & 1

    def fire(k, buf):
        return (
            pltpu.async_copy(z2_hbm.at[gat_v.at[k]], hrows.at[buf], sem),
            pltpu.async_copy(z2_hbm.at[gat_v.at[NCH + k]], trows.at[buf], sem),
        )

    lanes = lax.iota(jnp.int32, L)

    def compute(k, buf):
        def group(g, carry):
            ph = par_v[pl.ds(k * CH + g * L, L)] > 0
            pt = par_v[pl.ds(BPW + k * CH + g * L, L)] > 0
            rll = jnp.zeros((L,), jnp.float32)
            rlr = jnp.zeros((L,), jnp.float32)
            rrl = jnp.zeros((L,), jnp.float32)
            rrr = jnp.zeros((L,), jnp.float32)
            for j in range(L):
                r = g * L + j
                sll = jnp.zeros((L,), jnp.float32)
                slr = jnp.zeros((L,), jnp.float32)
                srl = jnp.zeros((L,), jnp.float32)
                srr = jnp.zeros((L,), jnp.float32)
                for c in range(D // L):
                    hlo = hrows[buf, r, pl.ds(c * L, L)]
                    hhi = hrows[buf, r, pl.ds(D + c * L, L)]
                    tlo = trows[buf, r, pl.ds(c * L, L)]
                    thi = trows[buf, r, pl.ds(D + c * L, L)]
                    sll = sll + hlo * tlo
                    slr = slr + hlo * thi
                    srl = srl + hhi * tlo
                    srr = srr + hhi * thi
                m = lanes == j
                rll = jnp.where(m, jnp.sum(sll), rll)
                rlr = jnp.where(m, jnp.sum(slr), rlr)
                rrl = jnp.where(m, jnp.sum(srl), rrl)
                rrr = jnp.where(m, jnp.sum(srr), rrr)
            res = jnp.where(ph, jnp.where(pt, rrr, rrl),
                            jnp.where(pt, rlr, rll))
            out_v[pl.ds(k * CH + g * L, L)] = res
            return carry

        lax.fori_loop(0, CH // L, group, 0)

    descs = {0: fire(0, 0)}
    for k in range(NCH):
        if k + 1 < NCH:
            descs[k + 1] = fire(k + 1, (k + 1) % 2)
        for d in descs.pop(k):
            d.wait()
        compute(k, k % 2)

    pltpu.sync_copy(out_v, out_hbm.at[pl.ds(base, BPW)])


def _decode(z2, h, t):
    return pl.kernel(
        _decoder_body,
        out_type=jax.ShapeDtypeStruct((B,), jnp.float32),
        mesh=_mesh(),
        compiler_params=pltpu.CompilerParams(needs_layout_passes=False),
        scratch_types=[
            pltpu.VMEM((2 * BPW,), jnp.int32),
            pltpu.VMEM((2 * NCH, CH), jnp.int32),
            pltpu.VMEM((2 * BPW,), jnp.int32),
            pltpu.VMEM((2, CH, 2 * D), jnp.float32),
            pltpu.VMEM((2, CH, 2 * D), jnp.float32),
            pltpu.VMEM((BPW,), jnp.float32),
            pltpu.SemaphoreType.DMA,
        ],
    )(z2, h, t)


def kernel(z, triples):
    h = triples[:, 0].astype(jnp.int32)
    t = triples[:, 2].astype(jnp.int32)
    ztail = z[NSLAB * SLABW:].reshape(-1)
    z2 = _transpose(z.T, ztail)
    return _decode(z2, h, t)
